# Initial kernel scaffold; baseline (speedup 1.0000x reference)
import jax
import jax.numpy as jnp
from jax.experimental import pallas as pl


def kernel(distance, edge_index, atom_types, graph_ids, emb, W1, b1, W2, b2, bn1_g, bn1_b, bn2_g, bn2_b, bn3_g, bn3_b, Wfc, bfc, Wout, bout):
    raise NotImplementedError("write your pallas kernel here")


# speedup vs baseline: 1.3385x; 1.3385x over previous
import functools

import jax
import jax.numpy as jnp
from jax import lax
from jax.experimental import pallas as pl
from jax.experimental.pallas import tpu as pltpu
from jax.experimental.pallas import tpu_sc as plsc

DIM = 64
NG = 64
CUTOFF = 12.0
NCONV = 3

NP = 50176
EP = 819200
NB = NP // 512
EB = 2048
NEB = EP // EB
HALF = NP // 2
SROWS = 25600
TRASH = HALF
ZR = 1600

_sc_mesh = lambda: plsc.VectorSubcoreMesh(core_axis_name="c", subcore_axis_name="s",
                                          num_cores=2, num_subcores=16)


def _sc_gather_add(pa, pb, src, dst):
    per_w = EP // 32
    C = 128
    nch = per_w // C

    @functools.partial(
        pl.kernel,
        out_type=jax.ShapeDtypeStruct((EP, DIM), jnp.float32),
        mesh=_sc_mesh(),
        scratch_types=[
            pltpu.VMEM((C,), jnp.int32),
            pltpu.VMEM((C,), jnp.int32),
            pltpu.VMEM((C, DIM), jnp.float32),
            pltpu.VMEM((C, DIM), jnp.float32),
            pltpu.SemaphoreType.DMA,
            pltpu.SemaphoreType.DMA,
        ],
        compiler_params=pltpu.CompilerParams(use_tc_tiling_on_sc=False),
    )
    def k(pa_hbm, pb_hbm, src_hbm, dst_hbm, g_hbm, ia, ib, ra, rb, sa, sb):
        wid = lax.axis_index("s") * 2 + lax.axis_index("c")
        base = wid * per_w

        @pl.loop(0, nch)
        def _(ci):
            off = base + ci * C
            pltpu.sync_copy(src_hbm.at[pl.ds(off, C)], ia)
            pltpu.sync_copy(dst_hbm.at[pl.ds(off, C)], ib)
            ca = pltpu.async_copy(pa_hbm.at[ia], ra, sa)
            cb = pltpu.async_copy(pb_hbm.at[ib], rb, sb)
            ca.wait()
            cb.wait()

            @pl.loop(0, C)
            def _(r):
                for col in range(0, DIM, 16):
                    s = pl.ds(col, 16)
                    ra[r, s] = ra[r, s] + rb[r, s]

            pltpu.sync_copy(ra, g_hbm.at[pl.ds(off, C)])

    return k(pa, pb, src, dst)


def _sc_scatter_add(h, dst, zrows):
    per_t = EP // 16
    C = 128
    nch = per_t // C
    wo = HALF // 16

    @functools.partial(
        pl.kernel,
        out_type=jax.ShapeDtypeStruct((NP, DIM), jnp.float32),
        mesh=_sc_mesh(),
        scratch_types=[
            pltpu.VMEM((C, DIM), jnp.float32),
            pltpu.VMEM((C,), jnp.int32),
            pltpu.VMEM((1, C), jnp.int32),
            pltpu.VMEM_SHARED((SROWS, DIM), jnp.float32),
            pltpu.SemaphoreType.DMA,
        ],
        compiler_params=pltpu.CompilerParams(use_tc_tiling_on_sc=False),
    )
    def k(h_hbm, dst_hbm, z_hbm, out_hbm, rows, di, li, acc, sem):
        c = lax.axis_index("c")
        s = lax.axis_index("s")
        pltpu.sync_copy(z_hbm, acc.at[pl.ds(s * ZR, ZR)])
        plsc.subcore_barrier()
        base_node = c * HALF

        @pl.loop(0, nch)
        def _(ci):
            off = s * per_t + ci * C
            pltpu.sync_copy(dst_hbm.at[pl.ds(off, C)], di)

            @pl.loop(0, C, step=16)
            def _(l):
                v = di[pl.ds(l, 16)] - base_node
                ok = (v >= 0) & (v < HALF)
                li[0, pl.ds(l, 16)] = jnp.where(ok, v, TRASH)

            pltpu.sync_copy(h_hbm.at[pl.ds(off, C)], rows)
            pltpu.sync_copy(rows, acc.at[li.at[0]], add=True)

        plsc.subcore_barrier()
        pltpu.sync_copy(acc.at[pl.ds(s * wo, wo)],
                        out_hbm.at[pl.ds(c * HALF + s * wo, wo)])

    return k(h, dst, zrows)


def _tc_first(at3, embp, wab):
    def body(at_ref, emb_ref, w_ref, node_ref, pa_ref, pb_ref):
        at = at_ref[0, 0, :]
        oh = (at[:, None] == lax.broadcasted_iota(jnp.int32, (512, 128), 1)
              ).astype(jnp.float32)
        node = jnp.dot(oh, emb_ref[...], preferred_element_type=jnp.float32)
        node_ref[...] = node
        proj = jnp.dot(node, w_ref[...], preferred_element_type=jnp.float32)
        pa_ref[...] = proj[:, :DIM]
        pb_ref[...] = proj[:, DIM:]

    return pl.pallas_call(
        body,
        grid=(NB,),
        in_specs=[
            pl.BlockSpec((1, 1, 512), lambda j: (j, 0, 0)),
            pl.BlockSpec((128, DIM), lambda j: (0, 0)),
            pl.BlockSpec((DIM, 2 * DIM), lambda j: (0, 0)),
        ],
        out_specs=[
            pl.BlockSpec((512, DIM), lambda j: (j, 0)),
            pl.BlockSpec((512, DIM), lambda j: (j, 0)),
            pl.BlockSpec((512, DIM), lambda j: (j, 0)),
        ],
        out_shape=[jax.ShapeDtypeStruct((NP, DIM), jnp.float32)] * 3,
    )(at3, embp, wab)


def _tc_update_proj(node_prev, new_node, wab, p3, n_true):
    def body(np_ref, nn_ref, w_ref, p3_ref, node_ref, pa_ref, pb_ref, acc):
        p = pl.program_id(0)
        j = pl.program_id(1)

        @pl.when((p == 0) & (j == 0))
        def _():
            acc[...] = jnp.zeros_like(acc)

        @pl.when(p == 0)
        def _():
            x = nn_ref[...]
            row = lax.broadcasted_iota(jnp.int32, (512, 1), 0) + j * 512
            xm = jnp.where(row < n_true, x, 0.0)
            acc[0:1, :] += jnp.sum(xm, axis=0, keepdims=True)
            acc[1:2, :] += jnp.sum(xm * xm, axis=0, keepdims=True)

        @pl.when(p == 1)
        def _():
            mu = acc[0:1, :] / n_true
            var = acc[1:2, :] / n_true - mu * mu
            rstd = lax.rsqrt(var + 1e-5)
            node = np_ref[...] + (nn_ref[...] - mu) * rstd * p3_ref[0:1, :] \
                + p3_ref[1:2, :]
            node_ref[...] = node
            proj = jnp.dot(node, w_ref[...], preferred_element_type=jnp.float32)
            pa_ref[...] = proj[:, :DIM]
            pb_ref[...] = proj[:, DIM:]

    return pl.pallas_call(
        body,
        grid=(2, NB),
        in_specs=[
            pl.BlockSpec((512, DIM), lambda p, j: (j, 0)),
            pl.BlockSpec((512, DIM), lambda p, j: (j, 0)),
            pl.BlockSpec((DIM, 2 * DIM), lambda p, j: (0, 0)),
            pl.BlockSpec((8, DIM), lambda p, j: (0, 0)),
        ],
        out_specs=[
            pl.BlockSpec((512, DIM), lambda p, j: (j, 0)),
            pl.BlockSpec((512, DIM), lambda p, j: (j, 0)),
            pl.BlockSpec((512, DIM), lambda p, j: (j, 0)),
        ],
        out_shape=[jax.ShapeDtypeStruct((NP, DIM), jnp.float32)] * 3,
        scratch_shapes=[pltpu.VMEM((8, DIM), jnp.float32)],
    )(node_prev, new_node, wab, p3)


def _tc_zstats(g, dist3, wc, pb, e_true):
    delta = CUTOFF / (NG - 1)

    def body(g_ref, d_ref, wc_ref, pb_ref, z_ref, st_ref):
        j = pl.program_id(0)
        d = d_ref[...]
        cent = lax.broadcasted_iota(jnp.int32, (1, NG), 1).astype(jnp.float32) * delta
        rbf = jnp.exp(-(((d - cent) / delta) ** 2))
        z = g_ref[...] + jnp.dot(rbf, wc_ref[...],
                                 preferred_element_type=jnp.float32) \
            + pb_ref[0:1, :]
        z_ref[...] = z
        row = lax.broadcasted_iota(jnp.int32, (EB, 1), 0) + j * EB
        zm = jnp.where(row < e_true, z, 0.0)

        @pl.when(j == 0)
        def _():
            st_ref[...] = jnp.zeros_like(st_ref)

        st_ref[0:1, :] += jnp.sum(zm, axis=0, keepdims=True)
        st_ref[1:2, :] += jnp.sum(zm * zm, axis=0, keepdims=True)

    return pl.pallas_call(
        body,
        grid=(NEB,),
        in_specs=[
            pl.BlockSpec((EB, DIM), lambda j: (j, 0)),
            pl.BlockSpec((EB, 1), lambda j: (j, 0)),
            pl.BlockSpec((NG, DIM), lambda j: (0, 0)),
            pl.BlockSpec((8, DIM), lambda j: (0, 0)),
        ],
        out_specs=[
            pl.BlockSpec((EB, DIM), lambda j: (j, 0)),
            pl.BlockSpec((8, DIM), lambda j: (0, 0)),
        ],
        out_shape=[
            jax.ShapeDtypeStruct((EP, DIM), jnp.float32),
            jax.ShapeDtypeStruct((8, DIM), jnp.float32),
        ],
    )(g, dist3, wc, pb)


def _f_of_z(z, st1_ref, pb_ref, e_true):
    mu = st1_ref[0:1, :] / e_true
    var = st1_ref[1:2, :] / e_true - mu * mu
    rstd = lax.rsqrt(var + 1e-5)
    return jax.nn.sigmoid((z - mu) * rstd * pb_ref[1:2, :] + pb_ref[2:3, :])


def _tc_fstats(z, st1, pb, e_true):
    def body(z_ref, st1_ref, pb_ref, st_ref):
        j = pl.program_id(0)
        f = _f_of_z(z_ref[...], st1_ref, pb_ref, e_true)
        row = lax.broadcasted_iota(jnp.int32, (EB, 1), 0) + j * EB
        fm = jnp.where(row < e_true, f, 0.0)

        @pl.when(j == 0)
        def _():
            st_ref[...] = jnp.zeros_like(st_ref)

        st_ref[0:1, :] += jnp.sum(fm, axis=0, keepdims=True)
        st_ref[1:2, :] += jnp.sum(fm * fm, axis=0, keepdims=True)

    return pl.pallas_call(
        body,
        grid=(NEB,),
        in_specs=[
            pl.BlockSpec((EB, DIM), lambda j: (j, 0)),
            pl.BlockSpec((8, DIM), lambda j: (0, 0)),
            pl.BlockSpec((8, DIM), lambda j: (0, 0)),
        ],
        out_specs=[pl.BlockSpec((8, DIM), lambda j: (0, 0))],
        out_shape=[jax.ShapeDtypeStruct((8, DIM), jnp.float32)],
    )(z, st1, pb)[0]


def _tc_h(z, st1, st2, pb, e_true):
    def body(z_ref, st1_ref, st2_ref, pb_ref, h_ref):
        j = pl.program_id(0)
        f = _f_of_z(z_ref[...], st1_ref, pb_ref, e_true)
        mu = st2_ref[0:1, :] / e_true
        var = st2_ref[1:2, :] / e_true - mu * mu
        rstd = lax.rsqrt(var + 1e-5)
        c = jax.nn.softplus((f - mu) * rstd * pb_ref[3:4, :] + pb_ref[4:5, :])
        h = f * c
        row = lax.broadcasted_iota(jnp.int32, (EB, 1), 0) + j * EB
        h_ref[...] = jnp.where(row < e_true, h, 0.0)

    return pl.pallas_call(
        body,
        grid=(NEB,),
        in_specs=[
            pl.BlockSpec((EB, DIM), lambda j: (j, 0)),
            pl.BlockSpec((8, DIM), lambda j: (0, 0)),
            pl.BlockSpec((8, DIM), lambda j: (0, 0)),
            pl.BlockSpec((8, DIM), lambda j: (0, 0)),
        ],
        out_specs=[pl.BlockSpec((EB, DIM), lambda j: (j, 0))],
        out_shape=[jax.ShapeDtypeStruct((EP, DIM), jnp.float32)],
    )(z, st1, st2, pb)[0]


def _tc_readout(node_prev, new_node, gid3, p3, wfc, fcaux, n_true, ngraph):
    def body(np_ref, nn_ref, gid_ref, p3_ref, wfc_ref, aux_ref, out_ref,
             acc, gsum, gcnt):
        p = pl.program_id(0)
        j = pl.program_id(1)

        @pl.when((p == 0) & (j == 0))
        def _():
            acc[...] = jnp.zeros_like(acc)
            gsum[...] = jnp.zeros_like(gsum)
            gcnt[...] = jnp.zeros_like(gcnt)

        @pl.when(p == 0)
        def _():
            x = nn_ref[...]
            row = lax.broadcasted_iota(jnp.int32, (512, 1), 0) + j * 512
            xm = jnp.where(row < n_true, x, 0.0)
            acc[0:1, :] += jnp.sum(xm, axis=0, keepdims=True)
            acc[1:2, :] += jnp.sum(xm * xm, axis=0, keepdims=True)

        @pl.when(p == 1)
        def _():
            mu = acc[0:1, :] / n_true
            var = acc[1:2, :] / n_true - mu * mu
            rstd = lax.rsqrt(var + 1e-5)
            node = np_ref[...] + (nn_ref[...] - mu) * rstd * p3_ref[0:1, :] \
                + p3_ref[1:2, :]
            gid = gid_ref[0, 0, :]
            oh = (gid[:, None] == lax.broadcasted_iota(
                jnp.int32, (512, ngraph), 1)).astype(jnp.float32)
            gsum[...] += lax.dot_general(
                oh, node, (((0,), (0,)), ((), ())),
                preferred_element_type=jnp.float32)
            gcnt[0:1, :] += jnp.sum(oh, axis=0, keepdims=True)

        @pl.when((p == 1) & (j == NB - 1))
        def _():
            cnt = jnp.transpose(gcnt[0:1, :], (1, 0))
            crys = gsum[...] / jnp.maximum(cnt, 1.0)
            a1 = jnp.dot(jax.nn.softplus(crys), wfc_ref[...],
                         preferred_element_type=jnp.float32) + aux_ref[0:1, :]
            a2 = jax.nn.softplus(a1)
            res = jnp.sum(a2 * aux_ref[1:2, :], axis=1, keepdims=True) \
                + aux_ref[2, 0]
            out_ref[...] = res

    return pl.pallas_call(
        body,
        grid=(2, NB),
        in_specs=[
            pl.BlockSpec((512, DIM), lambda p, j: (j, 0)),
            pl.BlockSpec((512, DIM), lambda p, j: (j, 0)),
            pl.BlockSpec((1, 1, 512), lambda p, j: (j, 0, 0)),
            pl.BlockSpec((8, DIM), lambda p, j: (0, 0)),
            pl.BlockSpec((DIM, 128), lambda p, j: (0, 0)),
            pl.BlockSpec((8, 128), lambda p, j: (0, 0)),
        ],
        out_specs=[pl.BlockSpec((ngraph, 1), lambda p, j: (0, 0))],
        out_shape=[jax.ShapeDtypeStruct((ngraph, 1), jnp.float32)],
        scratch_shapes=[
            pltpu.VMEM((8, DIM), jnp.float32),
            pltpu.VMEM((ngraph, DIM), jnp.float32),
            pltpu.VMEM((8, ngraph), jnp.float32),
        ],
    )(node_prev, new_node, gid3, p3, wfc, fcaux)[0]


def kernel(distance, edge_index, atom_types, graph_ids, emb, W1, b1, W2, b2,
           bn1_g, bn1_b, bn2_g, bn2_b, bn3_g, bn3_b, Wfc, bfc, Wout, bout):
    E = distance.shape[0]
    N = atom_types.shape[0]
    ngraph = 512
    e_true = float(E)
    n_true = float(N)

    src = jnp.pad(edge_index[0], (0, EP - E))
    dst = jnp.pad(edge_index[1], (0, EP - E))
    dist3 = jnp.pad(distance, (0, EP - E)).reshape(EP, 1)
    at3 = jnp.pad(atom_types, (0, NP - N)).reshape(NB, 1, 512)
    gid3 = jnp.pad(graph_ids, (0, NP - N), constant_values=ngraph
                   ).reshape(NB, 1, 512)
    embp = jnp.pad(emb, ((0, 128 - emb.shape[0]), (0, 0)))
    zrows = jnp.zeros((ZR, DIM), jnp.float32)
    fcaux = jnp.zeros((8, 128), jnp.float32)
    fcaux = fcaux.at[0, :].set(bfc).at[1, :].set(Wout[:, 0]).at[2, 0].set(bout[0])

    node = None
    new_node = None
    for i in range(NCONV):
        wab = jnp.concatenate([W1[i, :DIM, :], W1[i, DIM:2 * DIM, :]], axis=1)
        wc = W1[i, 2 * DIM:, :]
        pb = jnp.stack([b1[i], bn1_g[i], bn1_b[i], bn2_g[i], bn2_b[i],
                        jnp.zeros_like(b1[i]), jnp.zeros_like(b1[i]),
                        jnp.zeros_like(b1[i])])
        if i == 0:
            node, pa, pbj = _tc_first(at3, embp, wab)
        else:
            p3 = jnp.stack([bn3_g[i - 1], bn3_b[i - 1]] + [jnp.zeros_like(b1[0])] * 6)
            node, pa, pbj = _tc_update_proj(node, new_node, wab, p3, n_true)
        g = _sc_gather_add(pa, pbj, src, dst)
        z, st1 = _tc_zstats(g, dist3, wc, pb, e_true)
        st2 = _tc_fstats(z, st1, pb, e_true)
        h = _tc_h(z, st1, st2, pb, e_true)
        new_node = _sc_scatter_add(h, dst, zrows)

    p3 = jnp.stack([bn3_g[NCONV - 1], bn3_b[NCONV - 1]]
                   + [jnp.zeros_like(b1[0])] * 6)
    return _tc_readout(node, new_node, gid3, p3, Wfc, fcaux, n_true, ngraph)


# speedup vs baseline: 1.5001x; 1.1207x over previous
import functools

import jax
import jax.numpy as jnp
from jax import lax
from jax.experimental import pallas as pl
from jax.experimental.pallas import tpu as pltpu
from jax.experimental.pallas import tpu_sc as plsc

DIM = 64
NG = 64
CUTOFF = 12.0
NCONV = 3

NP = 50176
EP = 819200
NB = NP // 512
EB = 2048
NEB = EP // EB
HALF = NP // 2
SROWS = 25120
TRASH = HALF
ZR = 1570

_sc_mesh = lambda: plsc.VectorSubcoreMesh(core_axis_name="c", subcore_axis_name="s",
                                          num_cores=2, num_subcores=16)


def _sc_gather_add(pa, pb, src, dst):
    per_w = EP // 32
    CH = 256
    Q = CH // 128
    nch = per_w // CH

    @functools.partial(
        pl.kernel,
        out_type=jax.ShapeDtypeStruct((EP, DIM), jnp.float32),
        mesh=_sc_mesh(),
        scratch_types=[
            [pltpu.VMEM((CH,), jnp.int32)] * 2,
            [pltpu.VMEM((CH,), jnp.int32)] * 2,
            [pltpu.VMEM((CH, DIM), jnp.float32)] * 2,
            [pltpu.VMEM((CH, DIM), jnp.float32)] * 2,
            [pltpu.SemaphoreType.DMA] * 2,
            [pltpu.SemaphoreType.DMA] * 2,
            [pltpu.SemaphoreType.DMA] * 2,
        ],
        compiler_params=pltpu.CompilerParams(use_tc_tiling_on_sc=False),
    )
    def k(pa_hbm, pb_hbm, src_hbm, dst_hbm, g_hbm, ia, ib, ra, rb, si, sg, ss):
        wid = lax.axis_index("s") * 2 + lax.axis_index("c")
        base = wid * per_w

        def fire_idx(kb, ci):
            off = base + ci * CH
            pltpu.async_copy(src_hbm.at[pl.ds(off, CH)], ia[kb], si[kb])
            pltpu.async_copy(dst_hbm.at[pl.ds(off, CH)], ib[kb], si[kb])

        def wait_idx(kb, ci):
            off = base + ci * CH
            pltpu.make_async_copy(src_hbm.at[pl.ds(off, CH)], ia[kb], si[kb]).wait()
            pltpu.make_async_copy(dst_hbm.at[pl.ds(off, CH)], ib[kb], si[kb]).wait()

        def fire_gather(kb):
            for q in range(Q):
                s = pl.ds(q * 128, 128)
                pltpu.async_copy(pa_hbm.at[ia[kb].at[s]], ra[kb].at[s], sg[kb])
                pltpu.async_copy(pb_hbm.at[ib[kb].at[s]], rb[kb].at[s], sg[kb])

        def wait_gather(kb):
            for q in range(Q):
                s = pl.ds(q * 128, 128)
                pltpu.make_async_copy(pa_hbm.at[ia[kb].at[s]], ra[kb].at[s],
                                      sg[kb]).wait()
                pltpu.make_async_copy(pb_hbm.at[ib[kb].at[s]], rb[kb].at[s],
                                      sg[kb]).wait()

        def fire_store(kb, ci):
            off = base + ci * CH
            pltpu.async_copy(ra[kb], g_hbm.at[pl.ds(off, CH)], ss[kb])

        def wait_store(kb, ci):
            off = base + ci * CH
            pltpu.make_async_copy(ra[kb], g_hbm.at[pl.ds(off, CH)], ss[kb]).wait()

        fire_idx(0, 0)
        wait_idx(0, 0)
        fire_gather(0)
        fire_idx(1, 1)

        @pl.loop(0, nch, step=2)
        def _(ci):
            for u in range(2):
                kb = u
                ko = 1 - u
                ck = ci + u
                wait_gather(kb)

                @pl.when(ck + 2 < nch)
                def _():
                    fire_idx(kb, ck + 2)

                @pl.when(ck >= 1)
                def _():
                    wait_store(ko, ck - 1)

                @pl.when(ck + 1 < nch)
                def _():
                    wait_idx(ko, ck + 1)
                    fire_gather(ko)

                @pl.loop(0, CH, step=4)
                def _(r):
                    for v in range(4):
                        for colq in range(4):
                            s = pl.ds(colq * 16, 16)
                            ra[kb][r + v, s] = ra[kb][r + v, s] + rb[kb][r + v, s]

                fire_store(kb, ck)

        wait_store(1, nch - 1)

    return k(pa, pb, src, dst)


def _sc_scatter_add(h, dst, zrows):
    per_t = EP // 16
    C = 128
    nch = per_t // C
    wo = HALF // 16

    @functools.partial(
        pl.kernel,
        out_type=jax.ShapeDtypeStruct((NP, DIM), jnp.float32),
        mesh=_sc_mesh(),
        scratch_types=[
            [pltpu.VMEM((C, DIM), jnp.float32)] * 2,
            [pltpu.VMEM((C,), jnp.int32)] * 2,
            [pltpu.VMEM((C // 128, 128), jnp.int32)] * 2,
            pltpu.VMEM_SHARED((SROWS, DIM), jnp.float32),
            [pltpu.SemaphoreType.DMA] * 2,
            [pltpu.SemaphoreType.DMA] * 2,
        ],
        compiler_params=pltpu.CompilerParams(use_tc_tiling_on_sc=False),
    )
    def k(h_hbm, dst_hbm, z_hbm, out_hbm, rows, di, li, acc, sl, sa):
        c = lax.axis_index("c")
        s = lax.axis_index("s")
        QS = C // 128
        pltpu.sync_copy(z_hbm, acc.at[pl.ds(s * ZR, ZR)])
        plsc.subcore_barrier()
        base_node = c * HALF

        def fire_load(kb, ci):
            off = s * per_t + ci * C
            pltpu.async_copy(dst_hbm.at[pl.ds(off, C)], di[kb], sl[kb])
            pltpu.async_copy(h_hbm.at[pl.ds(off, C)], rows[kb], sl[kb])

        def wait_load(kb, ci):
            off = s * per_t + ci * C
            pltpu.make_async_copy(dst_hbm.at[pl.ds(off, C)], di[kb], sl[kb]).wait()
            pltpu.make_async_copy(h_hbm.at[pl.ds(off, C)], rows[kb], sl[kb]).wait()

        def fire_scatter(kb):
            for q in range(QS):
                pltpu.async_copy(rows[kb].at[pl.ds(q * 128, 128)],
                                 acc.at[li[kb].at[q]], sa[kb], add=True)

        def wait_scatter(kb):
            for q in range(QS):
                pltpu.make_async_copy(rows[kb].at[pl.ds(q * 128, 128)],
                                      acc.at[li[kb].at[q]], sa[kb]).wait()

        fire_load(0, 0)

        @pl.loop(0, nch, step=2)
        def _(ci):
            for u in range(2):
                kb = u
                ko = 1 - u
                ck = ci + u
                wait_load(kb, ck)
                for q in range(QS):
                    @pl.loop(0, 128, step=16)
                    def _(l):
                        v = di[kb][pl.ds(q * 128 + l, 16)] - base_node
                        ok = (v >= 0) & (v < HALF)
                        li[kb][q, pl.ds(l, 16)] = jnp.where(ok, v, TRASH)

                @pl.when(ck >= 1)
                def _():
                    wait_scatter(ko)

                @pl.when(ck + 1 < nch)
                def _():
                    fire_load(ko, ck + 1)

                fire_scatter(kb)

        wait_scatter(1)
        plsc.subcore_barrier()
        pltpu.sync_copy(acc.at[pl.ds(s * wo, wo)],
                        out_hbm.at[pl.ds(c * HALF + s * wo, wo)])

    return k(h, dst, zrows)


def _tc_first(at3, embp, wab):
    def body(at_ref, emb_ref, w_ref, node_ref, pa_ref, pb_ref):
        at = at_ref[0, 0, :]
        oh = (at[:, None] == lax.broadcasted_iota(jnp.int32, (512, 128), 1)
              ).astype(jnp.float32)
        node = jnp.dot(oh, emb_ref[...], preferred_element_type=jnp.float32)
        node_ref[...] = node
        proj = jnp.dot(node, w_ref[...], preferred_element_type=jnp.float32)
        pa_ref[...] = proj[:, :DIM]
        pb_ref[...] = proj[:, DIM:]

    return pl.pallas_call(
        body,
        grid=(NB,),
        in_specs=[
            pl.BlockSpec((1, 1, 512), lambda j: (j, 0, 0)),
            pl.BlockSpec((128, DIM), lambda j: (0, 0)),
            pl.BlockSpec((DIM, 2 * DIM), lambda j: (0, 0)),
        ],
        out_specs=[
            pl.BlockSpec((512, DIM), lambda j: (j, 0)),
            pl.BlockSpec((512, DIM), lambda j: (j, 0)),
            pl.BlockSpec((512, DIM), lambda j: (j, 0)),
        ],
        out_shape=[jax.ShapeDtypeStruct((NP, DIM), jnp.float32)] * 3,
    )(at3, embp, wab)


def _tc_update_proj(node_prev, new_node, wab, p3, n_true):
    def body(np_ref, nn_ref, w_ref, p3_ref, node_ref, pa_ref, pb_ref, acc):
        p = pl.program_id(0)
        j = pl.program_id(1)

        @pl.when((p == 0) & (j == 0))
        def _():
            acc[...] = jnp.zeros_like(acc)

        @pl.when(p == 0)
        def _():
            x = nn_ref[...]
            row = lax.broadcasted_iota(jnp.int32, (512, 1), 0) + j * 512
            xm = jnp.where(row < n_true, x, 0.0)
            acc[0:1, :] += jnp.sum(xm, axis=0, keepdims=True)
            acc[1:2, :] += jnp.sum(xm * xm, axis=0, keepdims=True)

        @pl.when(p == 1)
        def _():
            mu = acc[0:1, :] / n_true
            var = acc[1:2, :] / n_true - mu * mu
            rstd = lax.rsqrt(var + 1e-5)
            node = np_ref[...] + (nn_ref[...] - mu) * rstd * p3_ref[0:1, :] \
                + p3_ref[1:2, :]
            node_ref[...] = node
            proj = jnp.dot(node, w_ref[...], preferred_element_type=jnp.float32)
            pa_ref[...] = proj[:, :DIM]
            pb_ref[...] = proj[:, DIM:]

    return pl.pallas_call(
        body,
        grid=(2, NB),
        in_specs=[
            pl.BlockSpec((512, DIM), lambda p, j: (j, 0)),
            pl.BlockSpec((512, DIM), lambda p, j: (j, 0)),
            pl.BlockSpec((DIM, 2 * DIM), lambda p, j: (0, 0)),
            pl.BlockSpec((8, DIM), lambda p, j: (0, 0)),
        ],
        out_specs=[
            pl.BlockSpec((512, DIM), lambda p, j: (j, 0)),
            pl.BlockSpec((512, DIM), lambda p, j: (j, 0)),
            pl.BlockSpec((512, DIM), lambda p, j: (j, 0)),
        ],
        out_shape=[jax.ShapeDtypeStruct((NP, DIM), jnp.float32)] * 3,
        scratch_shapes=[pltpu.VMEM((8, DIM), jnp.float32)],
    )(node_prev, new_node, wab, p3)


def _tc_zstats(g, dist3, wc, pb, e_true):
    delta = CUTOFF / (NG - 1)

    def body(g_ref, d_ref, wc_ref, pb_ref, z_ref, st_ref):
        j = pl.program_id(0)
        d = d_ref[...]
        cent = lax.broadcasted_iota(jnp.int32, (1, NG), 1).astype(jnp.float32) * delta
        rbf = jnp.exp(-(((d - cent) / delta) ** 2))
        z = g_ref[...] + jnp.dot(rbf, wc_ref[...],
                                 preferred_element_type=jnp.float32) \
            + pb_ref[0:1, :]
        z_ref[...] = z
        row = lax.broadcasted_iota(jnp.int32, (EB, 1), 0) + j * EB
        zm = jnp.where(row < e_true, z, 0.0)

        @pl.when(j == 0)
        def _():
            st_ref[...] = jnp.zeros_like(st_ref)

        st_ref[0:1, :] += jnp.sum(zm, axis=0, keepdims=True)
        st_ref[1:2, :] += jnp.sum(zm * zm, axis=0, keepdims=True)

    return pl.pallas_call(
        body,
        grid=(NEB,),
        in_specs=[
            pl.BlockSpec((EB, DIM), lambda j: (j, 0)),
            pl.BlockSpec((EB, 1), lambda j: (j, 0)),
            pl.BlockSpec((NG, DIM), lambda j: (0, 0)),
            pl.BlockSpec((8, DIM), lambda j: (0, 0)),
        ],
        out_specs=[
            pl.BlockSpec((EB, DIM), lambda j: (j, 0)),
            pl.BlockSpec((8, DIM), lambda j: (0, 0)),
        ],
        out_shape=[
            jax.ShapeDtypeStruct((EP, DIM), jnp.float32),
            jax.ShapeDtypeStruct((8, DIM), jnp.float32),
        ],
    )(g, dist3, wc, pb)


def _f_of_z(z, st1_ref, pb_ref, e_true):
    mu = st1_ref[0:1, :] / e_true
    var = st1_ref[1:2, :] / e_true - mu * mu
    rstd = lax.rsqrt(var + 1e-5)
    return jax.nn.sigmoid((z - mu) * rstd * pb_ref[1:2, :] + pb_ref[2:3, :])


def _tc_fstats(z, st1, pb, e_true):
    def body(z_ref, st1_ref, pb_ref, st_ref):
        j = pl.program_id(0)
        f = _f_of_z(z_ref[...], st1_ref, pb_ref, e_true)
        row = lax.broadcasted_iota(jnp.int32, (EB, 1), 0) + j * EB
        fm = jnp.where(row < e_true, f, 0.0)

        @pl.when(j == 0)
        def _():
            st_ref[...] = jnp.zeros_like(st_ref)

        st_ref[0:1, :] += jnp.sum(fm, axis=0, keepdims=True)
        st_ref[1:2, :] += jnp.sum(fm * fm, axis=0, keepdims=True)

    return pl.pallas_call(
        body,
        grid=(NEB,),
        in_specs=[
            pl.BlockSpec((EB, DIM), lambda j: (j, 0)),
            pl.BlockSpec((8, DIM), lambda j: (0, 0)),
            pl.BlockSpec((8, DIM), lambda j: (0, 0)),
        ],
        out_specs=[pl.BlockSpec((8, DIM), lambda j: (0, 0))],
        out_shape=[jax.ShapeDtypeStruct((8, DIM), jnp.float32)],
    )(z, st1, pb)[0]


def _tc_h(z, st1, st2, pb, e_true):
    def body(z_ref, st1_ref, st2_ref, pb_ref, h_ref):
        j = pl.program_id(0)
        f = _f_of_z(z_ref[...], st1_ref, pb_ref, e_true)
        mu = st2_ref[0:1, :] / e_true
        var = st2_ref[1:2, :] / e_true - mu * mu
        rstd = lax.rsqrt(var + 1e-5)
        c = jax.nn.softplus((f - mu) * rstd * pb_ref[3:4, :] + pb_ref[4:5, :])
        h = f * c
        row = lax.broadcasted_iota(jnp.int32, (EB, 1), 0) + j * EB
        h_ref[...] = jnp.where(row < e_true, h, 0.0)

    return pl.pallas_call(
        body,
        grid=(NEB,),
        in_specs=[
            pl.BlockSpec((EB, DIM), lambda j: (j, 0)),
            pl.BlockSpec((8, DIM), lambda j: (0, 0)),
            pl.BlockSpec((8, DIM), lambda j: (0, 0)),
            pl.BlockSpec((8, DIM), lambda j: (0, 0)),
        ],
        out_specs=[pl.BlockSpec((EB, DIM), lambda j: (j, 0))],
        out_shape=[jax.ShapeDtypeStruct((EP, DIM), jnp.float32)],
    )(z, st1, st2, pb)[0]


def _tc_readout(node_prev, new_node, gid3, p3, wfc, fcaux, n_true, ngraph):
    def body(np_ref, nn_ref, gid_ref, p3_ref, wfc_ref, aux_ref, out_ref,
             acc, gsum, gcnt):
        p = pl.program_id(0)
        j = pl.program_id(1)

        @pl.when((p == 0) & (j == 0))
        def _():
            acc[...] = jnp.zeros_like(acc)
            gsum[...] = jnp.zeros_like(gsum)
            gcnt[...] = jnp.zeros_like(gcnt)

        @pl.when(p == 0)
        def _():
            x = nn_ref[...]
            row = lax.broadcasted_iota(jnp.int32, (512, 1), 0) + j * 512
            xm = jnp.where(row < n_true, x, 0.0)
            acc[0:1, :] += jnp.sum(xm, axis=0, keepdims=True)
            acc[1:2, :] += jnp.sum(xm * xm, axis=0, keepdims=True)

        @pl.when(p == 1)
        def _():
            mu = acc[0:1, :] / n_true
            var = acc[1:2, :] / n_true - mu * mu
            rstd = lax.rsqrt(var + 1e-5)
            node = np_ref[...] + (nn_ref[...] - mu) * rstd * p3_ref[0:1, :] \
                + p3_ref[1:2, :]
            gid = gid_ref[0, 0, :]
            oh = (gid[:, None] == lax.broadcasted_iota(
                jnp.int32, (512, ngraph), 1)).astype(jnp.float32)
            gsum[...] += lax.dot_general(
                oh, node, (((0,), (0,)), ((), ())),
                preferred_element_type=jnp.float32)
            gcnt[0:1, :] += jnp.sum(oh, axis=0, keepdims=True)

        @pl.when((p == 1) & (j == NB - 1))
        def _():
            cnt = jnp.transpose(gcnt[0:1, :], (1, 0))
            crys = gsum[...] / jnp.maximum(cnt, 1.0)
            a1 = jnp.dot(jax.nn.softplus(crys), wfc_ref[...],
                         preferred_element_type=jnp.float32) + aux_ref[0:1, :]
            a2 = jax.nn.softplus(a1)
            res = jnp.sum(a2 * aux_ref[1:2, :], axis=1, keepdims=True) \
                + aux_ref[2, 0]
            out_ref[...] = res

    return pl.pallas_call(
        body,
        grid=(2, NB),
        in_specs=[
            pl.BlockSpec((512, DIM), lambda p, j: (j, 0)),
            pl.BlockSpec((512, DIM), lambda p, j: (j, 0)),
            pl.BlockSpec((1, 1, 512), lambda p, j: (j, 0, 0)),
            pl.BlockSpec((8, DIM), lambda p, j: (0, 0)),
            pl.BlockSpec((DIM, 128), lambda p, j: (0, 0)),
            pl.BlockSpec((8, 128), lambda p, j: (0, 0)),
        ],
        out_specs=[pl.BlockSpec((ngraph, 1), lambda p, j: (0, 0))],
        out_shape=[jax.ShapeDtypeStruct((ngraph, 1), jnp.float32)],
        scratch_shapes=[
            pltpu.VMEM((8, DIM), jnp.float32),
            pltpu.VMEM((ngraph, DIM), jnp.float32),
            pltpu.VMEM((8, ngraph), jnp.float32),
        ],
    )(node_prev, new_node, gid3, p3, wfc, fcaux)[0]


def kernel(distance, edge_index, atom_types, graph_ids, emb, W1, b1, W2, b2,
           bn1_g, bn1_b, bn2_g, bn2_b, bn3_g, bn3_b, Wfc, bfc, Wout, bout):
    E = distance.shape[0]
    N = atom_types.shape[0]
    ngraph = 512
    e_true = float(E)
    n_true = float(N)

    src = jnp.pad(edge_index[0], (0, EP - E))
    dst = jnp.pad(edge_index[1], (0, EP - E))
    dist3 = jnp.pad(distance, (0, EP - E)).reshape(EP, 1)
    at3 = jnp.pad(atom_types, (0, NP - N)).reshape(NB, 1, 512)
    gid3 = jnp.pad(graph_ids, (0, NP - N), constant_values=ngraph
                   ).reshape(NB, 1, 512)
    embp = jnp.pad(emb, ((0, 128 - emb.shape[0]), (0, 0)))
    zrows = jnp.zeros((ZR, DIM), jnp.float32)
    fcaux = jnp.zeros((8, 128), jnp.float32)
    fcaux = fcaux.at[0, :].set(bfc).at[1, :].set(Wout[:, 0]).at[2, 0].set(bout[0])

    node = None
    new_node = None
    for i in range(NCONV):
        wab = jnp.concatenate([W1[i, :DIM, :], W1[i, DIM:2 * DIM, :]], axis=1)
        wc = W1[i, 2 * DIM:, :]
        pb = jnp.stack([b1[i], bn1_g[i], bn1_b[i], bn2_g[i], bn2_b[i],
                        jnp.zeros_like(b1[i]), jnp.zeros_like(b1[i]),
                        jnp.zeros_like(b1[i])])
        if i == 0:
            node, pa, pbj = _tc_first(at3, embp, wab)
        else:
            p3 = jnp.stack([bn3_g[i - 1], bn3_b[i - 1]] + [jnp.zeros_like(b1[0])] * 6)
            node, pa, pbj = _tc_update_proj(node, new_node, wab, p3, n_true)
        g = _sc_gather_add(pa, pbj, src, dst)
        z, st1 = _tc_zstats(g, dist3, wc, pb, e_true)
        st2 = _tc_fstats(z, st1, pb, e_true)
        h = _tc_h(z, st1, st2, pb, e_true)
        new_node = _sc_scatter_add(h, dst, zrows)

    p3 = jnp.stack([bn3_g[NCONV - 1], bn3_b[NCONV - 1]]
                   + [jnp.zeros_like(b1[0])] * 6)
    return _tc_readout(node, new_node, gid3, p3, Wfc, fcaux, n_true, ngraph)


# speedup vs baseline: 1.5759x; 1.0505x over previous
import functools

import jax
import jax.numpy as jnp
from jax import lax
from jax.experimental import pallas as pl
from jax.experimental.pallas import tpu as pltpu
from jax.experimental.pallas import tpu_sc as plsc

DIM = 64
NG = 64
CUTOFF = 12.0
NCONV = 3

NP = 50176
EP = 819200
NB = NP // 512
EB = 2048
NEB = EP // EB
HALF = NP // 2
SROWS = 25120
TRASH = HALF
ZR = 1570

_sc_mesh = lambda: plsc.VectorSubcoreMesh(core_axis_name="c", subcore_axis_name="s",
                                          num_cores=2, num_subcores=16)


def _sc_gather_add(pa, pb, src, dst):
    per_w = EP // 32
    CH = 512
    Q = CH // 128
    nch = per_w // CH

    @functools.partial(
        pl.kernel,
        out_type=jax.ShapeDtypeStruct((EP, DIM), jnp.bfloat16),
        mesh=_sc_mesh(),
        scratch_types=[
            [pltpu.VMEM((CH,), jnp.int32)] * 2,
            [pltpu.VMEM((CH,), jnp.int32)] * 2,
            [pltpu.VMEM((CH, DIM), jnp.bfloat16)] * 2,
            [pltpu.VMEM((CH, DIM), jnp.bfloat16)] * 2,
            [pltpu.SemaphoreType.DMA] * 2,
            [pltpu.SemaphoreType.DMA] * 2,
            [pltpu.SemaphoreType.DMA] * 2,
        ],
        compiler_params=pltpu.CompilerParams(use_tc_tiling_on_sc=False),
    )
    def k(pa_hbm, pb_hbm, src_hbm, dst_hbm, g_hbm, ia, ib, ra, rb, si, sg, ss):
        wid = lax.axis_index("s") * 2 + lax.axis_index("c")
        base = wid * per_w

        def fire_idx(kb, ci):
            off = base + ci * CH
            pltpu.async_copy(src_hbm.at[pl.ds(off, CH)], ia[kb], si[kb])
            pltpu.async_copy(dst_hbm.at[pl.ds(off, CH)], ib[kb], si[kb])

        def wait_idx(kb, ci):
            off = base + ci * CH
            pltpu.make_async_copy(src_hbm.at[pl.ds(off, CH)], ia[kb], si[kb]).wait()
            pltpu.make_async_copy(dst_hbm.at[pl.ds(off, CH)], ib[kb], si[kb]).wait()

        def fire_gather(kb):
            for q in range(Q):
                s = pl.ds(q * 128, 128)
                pltpu.async_copy(pa_hbm.at[ia[kb].at[s]], ra[kb].at[s], sg[kb])
                pltpu.async_copy(pb_hbm.at[ib[kb].at[s]], rb[kb].at[s], sg[kb])

        def wait_gather(kb):
            for q in range(Q):
                s = pl.ds(q * 128, 128)
                pltpu.make_async_copy(pa_hbm.at[ia[kb].at[s]], ra[kb].at[s],
                                      sg[kb]).wait()
                pltpu.make_async_copy(pb_hbm.at[ib[kb].at[s]], rb[kb].at[s],
                                      sg[kb]).wait()

        def fire_store(kb, ci):
            off = base + ci * CH
            pltpu.async_copy(ra[kb], g_hbm.at[pl.ds(off, CH)], ss[kb])

        def wait_store(kb, ci):
            off = base + ci * CH
            pltpu.make_async_copy(ra[kb], g_hbm.at[pl.ds(off, CH)], ss[kb]).wait()

        fire_idx(0, 0)
        wait_idx(0, 0)
        fire_gather(0)
        fire_idx(1, 1)

        @pl.loop(0, nch, step=2)
        def _(ci):
            for u in range(2):
                kb = u
                ko = 1 - u
                ck = ci + u
                wait_gather(kb)

                @pl.when(ck + 2 < nch)
                def _():
                    fire_idx(kb, ck + 2)

                @pl.when(ck >= 1)
                def _():
                    wait_store(ko, ck - 1)

                @pl.when(ck + 1 < nch)
                def _():
                    wait_idx(ko, ck + 1)
                    fire_gather(ko)

                @pl.loop(0, CH, step=4)
                def _(r):
                    for v in range(4):
                        for colq in range(2):
                            s = pl.ds(colq * 32, 32)
                            ra[kb][r + v, s] = ra[kb][r + v, s] + rb[kb][r + v, s]

                fire_store(kb, ck)

        wait_store(1, nch - 1)

    return k(pa, pb, src, dst)


def _sc_scatter_add(h, dst, zrows):
    per_t = EP // 16
    C = 128
    nch = per_t // C
    wo = HALF // 16

    @functools.partial(
        pl.kernel,
        out_type=jax.ShapeDtypeStruct((NP, DIM), jnp.float32),
        mesh=_sc_mesh(),
        scratch_types=[
            [pltpu.VMEM((C, DIM), jnp.float32)] * 2,
            [pltpu.VMEM((C,), jnp.int32)] * 2,
            [pltpu.VMEM((C // 128, 128), jnp.int32)] * 2,
            pltpu.VMEM_SHARED((SROWS, DIM), jnp.float32),
            [pltpu.SemaphoreType.DMA] * 2,
            [pltpu.SemaphoreType.DMA] * 2,
        ],
        compiler_params=pltpu.CompilerParams(use_tc_tiling_on_sc=False),
    )
    def k(h_hbm, dst_hbm, z_hbm, out_hbm, rows, di, li, acc, sl, sa):
        c = lax.axis_index("c")
        s = lax.axis_index("s")
        QS = C // 128
        pltpu.sync_copy(z_hbm, acc.at[pl.ds(s * ZR, ZR)])
        plsc.subcore_barrier()
        base_node = c * HALF

        def fire_load(kb, ci):
            off = s * per_t + ci * C
            pltpu.async_copy(dst_hbm.at[pl.ds(off, C)], di[kb], sl[kb])
            pltpu.async_copy(h_hbm.at[pl.ds(off, C)], rows[kb], sl[kb])

        def wait_load(kb, ci):
            off = s * per_t + ci * C
            pltpu.make_async_copy(dst_hbm.at[pl.ds(off, C)], di[kb], sl[kb]).wait()
            pltpu.make_async_copy(h_hbm.at[pl.ds(off, C)], rows[kb], sl[kb]).wait()

        def fire_scatter(kb):
            for q in range(QS):
                pltpu.async_copy(rows[kb].at[pl.ds(q * 128, 128)],
                                 acc.at[li[kb].at[q]], sa[kb], add=True)

        def wait_scatter(kb):
            for q in range(QS):
                pltpu.make_async_copy(rows[kb].at[pl.ds(q * 128, 128)],
                                      acc.at[li[kb].at[q]], sa[kb]).wait()

        fire_load(0, 0)

        @pl.loop(0, nch, step=2)
        def _(ci):
            for u in range(2):
                kb = u
                ko = 1 - u
                ck = ci + u
                wait_load(kb, ck)
                for q in range(QS):
                    @pl.loop(0, 128, step=16)
                    def _(l):
                        v = di[kb][pl.ds(q * 128 + l, 16)] - base_node
                        ok = (v >= 0) & (v < HALF)
                        li[kb][q, pl.ds(l, 16)] = jnp.where(ok, v, TRASH)

                @pl.when(ck >= 1)
                def _():
                    wait_scatter(ko)

                @pl.when(ck + 1 < nch)
                def _():
                    fire_load(ko, ck + 1)

                fire_scatter(kb)

        wait_scatter(1)
        plsc.subcore_barrier()
        pltpu.sync_copy(acc.at[pl.ds(s * wo, wo)],
                        out_hbm.at[pl.ds(c * HALF + s * wo, wo)])

    return k(h, dst, zrows)


def _tc_first(at3, embp, wab):
    def body(at_ref, emb_ref, w_ref, node_ref, pa_ref, pb_ref):
        at = at_ref[0, 0, :]
        oh = (at[:, None] == lax.broadcasted_iota(jnp.int32, (512, 128), 1)
              ).astype(jnp.float32)
        node = jnp.dot(oh, emb_ref[...], preferred_element_type=jnp.float32)
        node_ref[...] = node
        proj = jnp.dot(node, w_ref[...], preferred_element_type=jnp.float32)
        pa_ref[...] = proj[:, :DIM].astype(jnp.bfloat16)
        pb_ref[...] = proj[:, DIM:].astype(jnp.bfloat16)

    return pl.pallas_call(
        body,
        grid=(NB,),
        in_specs=[
            pl.BlockSpec((1, 1, 512), lambda j: (j, 0, 0)),
            pl.BlockSpec((128, DIM), lambda j: (0, 0)),
            pl.BlockSpec((DIM, 2 * DIM), lambda j: (0, 0)),
        ],
        out_specs=[
            pl.BlockSpec((512, DIM), lambda j: (j, 0)),
            pl.BlockSpec((512, DIM), lambda j: (j, 0)),
            pl.BlockSpec((512, DIM), lambda j: (j, 0)),
        ],
        out_shape=[
            jax.ShapeDtypeStruct((NP, DIM), jnp.float32),
            jax.ShapeDtypeStruct((NP, DIM), jnp.bfloat16),
            jax.ShapeDtypeStruct((NP, DIM), jnp.bfloat16),
        ],
    )(at3, embp, wab)


def _tc_update_proj(node_prev, new_node, wab, p3, n_true):
    def body(np_ref, nn_ref, w_ref, p3_ref, node_ref, pa_ref, pb_ref, acc):
        p = pl.program_id(0)
        j = pl.program_id(1)

        @pl.when((p == 0) & (j == 0))
        def _():
            acc[...] = jnp.zeros_like(acc)

        @pl.when(p == 0)
        def _():
            x = nn_ref[...]
            row = lax.broadcasted_iota(jnp.int32, (512, 1), 0) + j * 512
            xm = jnp.where(row < n_true, x, 0.0)
            acc[0:1, :] += jnp.sum(xm, axis=0, keepdims=True)
            acc[1:2, :] += jnp.sum(xm * xm, axis=0, keepdims=True)

        @pl.when(p == 1)
        def _():
            mu = acc[0:1, :] / n_true
            var = acc[1:2, :] / n_true - mu * mu
            rstd = lax.rsqrt(var + 1e-5)
            node = np_ref[...] + (nn_ref[...] - mu) * rstd * p3_ref[0:1, :] \
                + p3_ref[1:2, :]
            node_ref[...] = node
            proj = jnp.dot(node, w_ref[...], preferred_element_type=jnp.float32)
            pa_ref[...] = proj[:, :DIM].astype(jnp.bfloat16)
            pb_ref[...] = proj[:, DIM:].astype(jnp.bfloat16)

    return pl.pallas_call(
        body,
        grid=(2, NB),
        in_specs=[
            pl.BlockSpec((512, DIM), lambda p, j: (j, 0)),
            pl.BlockSpec((512, DIM), lambda p, j: (j, 0)),
            pl.BlockSpec((DIM, 2 * DIM), lambda p, j: (0, 0)),
            pl.BlockSpec((8, DIM), lambda p, j: (0, 0)),
        ],
        out_specs=[
            pl.BlockSpec((512, DIM), lambda p, j: (j, 0)),
            pl.BlockSpec((512, DIM), lambda p, j: (j, 0)),
            pl.BlockSpec((512, DIM), lambda p, j: (j, 0)),
        ],
        out_shape=[
            jax.ShapeDtypeStruct((NP, DIM), jnp.float32),
            jax.ShapeDtypeStruct((NP, DIM), jnp.bfloat16),
            jax.ShapeDtypeStruct((NP, DIM), jnp.bfloat16),
        ],
        scratch_shapes=[pltpu.VMEM((8, DIM), jnp.float32)],
    )(node_prev, new_node, wab, p3)


def _tc_zstats(g, dist3, wc, pb, e_true):
    delta = CUTOFF / (NG - 1)

    def body(g_ref, d_ref, wc_ref, pb_ref, z_ref, st_ref):
        j = pl.program_id(0)
        d = d_ref[...]
        cent = lax.broadcasted_iota(jnp.int32, (1, NG), 1).astype(jnp.float32) * delta
        rbf = jnp.exp(-(((d - cent) / delta) ** 2))
        z = g_ref[...].astype(jnp.float32) + jnp.dot(rbf, wc_ref[...],
                                 preferred_element_type=jnp.float32) \
            + pb_ref[0:1, :]
        z_ref[...] = z
        row = lax.broadcasted_iota(jnp.int32, (EB, 1), 0) + j * EB
        zm = jnp.where(row < e_true, z, 0.0)

        @pl.when(j == 0)
        def _():
            st_ref[...] = jnp.zeros_like(st_ref)

        st_ref[0:1, :] += jnp.sum(zm, axis=0, keepdims=True)
        st_ref[1:2, :] += jnp.sum(zm * zm, axis=0, keepdims=True)

    return pl.pallas_call(
        body,
        grid=(NEB,),
        in_specs=[
            pl.BlockSpec((EB, DIM), lambda j: (j, 0)),
            pl.BlockSpec((EB, 1), lambda j: (j, 0)),
            pl.BlockSpec((NG, DIM), lambda j: (0, 0)),
            pl.BlockSpec((8, DIM), lambda j: (0, 0)),
        ],
        out_specs=[
            pl.BlockSpec((EB, DIM), lambda j: (j, 0)),
            pl.BlockSpec((8, DIM), lambda j: (0, 0)),
        ],
        out_shape=[
            jax.ShapeDtypeStruct((EP, DIM), jnp.float32),
            jax.ShapeDtypeStruct((8, DIM), jnp.float32),
        ],
    )(g, dist3, wc, pb)


def _f_of_z(z, st1_ref, pb_ref, e_true):
    mu = st1_ref[0:1, :] / e_true
    var = st1_ref[1:2, :] / e_true - mu * mu
    rstd = lax.rsqrt(var + 1e-5)
    return jax.nn.sigmoid((z - mu) * rstd * pb_ref[1:2, :] + pb_ref[2:3, :])


def _tc_fstats(z, st1, pb, e_true):
    def body(z_ref, st1_ref, pb_ref, st_ref):
        j = pl.program_id(0)
        f = _f_of_z(z_ref[...], st1_ref, pb_ref, e_true)
        row = lax.broadcasted_iota(jnp.int32, (EB, 1), 0) + j * EB
        fm = jnp.where(row < e_true, f, 0.0)

        @pl.when(j == 0)
        def _():
            st_ref[...] = jnp.zeros_like(st_ref)

        st_ref[0:1, :] += jnp.sum(fm, axis=0, keepdims=True)
        st_ref[1:2, :] += jnp.sum(fm * fm, axis=0, keepdims=True)

    return pl.pallas_call(
        body,
        grid=(NEB,),
        in_specs=[
            pl.BlockSpec((EB, DIM), lambda j: (j, 0)),
            pl.BlockSpec((8, DIM), lambda j: (0, 0)),
            pl.BlockSpec((8, DIM), lambda j: (0, 0)),
        ],
        out_specs=[pl.BlockSpec((8, DIM), lambda j: (0, 0))],
        out_shape=[jax.ShapeDtypeStruct((8, DIM), jnp.float32)],
    )(z, st1, pb)[0]


def _tc_h(z, st1, st2, pb, e_true):
    def body(z_ref, st1_ref, st2_ref, pb_ref, h_ref):
        j = pl.program_id(0)
        f = _f_of_z(z_ref[...], st1_ref, pb_ref, e_true)
        mu = st2_ref[0:1, :] / e_true
        var = st2_ref[1:2, :] / e_true - mu * mu
        rstd = lax.rsqrt(var + 1e-5)
        c = jax.nn.softplus((f - mu) * rstd * pb_ref[3:4, :] + pb_ref[4:5, :])
        h = f * c
        row = lax.broadcasted_iota(jnp.int32, (EB, 1), 0) + j * EB
        h_ref[...] = jnp.where(row < e_true, h, 0.0)

    return pl.pallas_call(
        body,
        grid=(NEB,),
        in_specs=[
            pl.BlockSpec((EB, DIM), lambda j: (j, 0)),
            pl.BlockSpec((8, DIM), lambda j: (0, 0)),
            pl.BlockSpec((8, DIM), lambda j: (0, 0)),
            pl.BlockSpec((8, DIM), lambda j: (0, 0)),
        ],
        out_specs=[pl.BlockSpec((EB, DIM), lambda j: (j, 0))],
        out_shape=[jax.ShapeDtypeStruct((EP, DIM), jnp.float32)],
    )(z, st1, st2, pb)[0]


def _tc_readout(node_prev, new_node, gid3, p3, wfc, fcaux, n_true, ngraph):
    def body(np_ref, nn_ref, gid_ref, p3_ref, wfc_ref, aux_ref, out_ref,
             acc, gsum, gcnt):
        p = pl.program_id(0)
        j = pl.program_id(1)

        @pl.when((p == 0) & (j == 0))
        def _():
            acc[...] = jnp.zeros_like(acc)
            gsum[...] = jnp.zeros_like(gsum)
            gcnt[...] = jnp.zeros_like(gcnt)

        @pl.when(p == 0)
        def _():
            x = nn_ref[...]
            row = lax.broadcasted_iota(jnp.int32, (512, 1), 0) + j * 512
            xm = jnp.where(row < n_true, x, 0.0)
            acc[0:1, :] += jnp.sum(xm, axis=0, keepdims=True)
            acc[1:2, :] += jnp.sum(xm * xm, axis=0, keepdims=True)

        @pl.when(p == 1)
        def _():
            mu = acc[0:1, :] / n_true
            var = acc[1:2, :] / n_true - mu * mu
            rstd = lax.rsqrt(var + 1e-5)
            node = np_ref[...] + (nn_ref[...] - mu) * rstd * p3_ref[0:1, :] \
                + p3_ref[1:2, :]
            gid = gid_ref[0, 0, :]
            oh = (gid[:, None] == lax.broadcasted_iota(
                jnp.int32, (512, ngraph), 1)).astype(jnp.float32)
            gsum[...] += lax.dot_general(
                oh, node, (((0,), (0,)), ((), ())),
                preferred_element_type=jnp.float32)
            gcnt[0:1, :] += jnp.sum(oh, axis=0, keepdims=True)

        @pl.when((p == 1) & (j == NB - 1))
        def _():
            cnt = jnp.transpose(gcnt[0:1, :], (1, 0))
            crys = gsum[...] / jnp.maximum(cnt, 1.0)
            a1 = jnp.dot(jax.nn.softplus(crys), wfc_ref[...],
                         preferred_element_type=jnp.float32) + aux_ref[0:1, :]
            a2 = jax.nn.softplus(a1)
            res = jnp.sum(a2 * aux_ref[1:2, :], axis=1, keepdims=True) \
                + aux_ref[2, 0]
            out_ref[...] = res

    return pl.pallas_call(
        body,
        grid=(2, NB),
        in_specs=[
            pl.BlockSpec((512, DIM), lambda p, j: (j, 0)),
            pl.BlockSpec((512, DIM), lambda p, j: (j, 0)),
            pl.BlockSpec((1, 1, 512), lambda p, j: (j, 0, 0)),
            pl.BlockSpec((8, DIM), lambda p, j: (0, 0)),
            pl.BlockSpec((DIM, 128), lambda p, j: (0, 0)),
            pl.BlockSpec((8, 128), lambda p, j: (0, 0)),
        ],
        out_specs=[pl.BlockSpec((ngraph, 1), lambda p, j: (0, 0))],
        out_shape=[jax.ShapeDtypeStruct((ngraph, 1), jnp.float32)],
        scratch_shapes=[
            pltpu.VMEM((8, DIM), jnp.float32),
            pltpu.VMEM((ngraph, DIM), jnp.float32),
            pltpu.VMEM((8, ngraph), jnp.float32),
        ],
    )(node_prev, new_node, gid3, p3, wfc, fcaux)[0]


def kernel(distance, edge_index, atom_types, graph_ids, emb, W1, b1, W2, b2,
           bn1_g, bn1_b, bn2_g, bn2_b, bn3_g, bn3_b, Wfc, bfc, Wout, bout):
    E = distance.shape[0]
    N = atom_types.shape[0]
    ngraph = 512
    e_true = float(E)
    n_true = float(N)

    src = jnp.pad(edge_index[0], (0, EP - E))
    dst = jnp.pad(edge_index[1], (0, EP - E))
    dist3 = jnp.pad(distance, (0, EP - E)).reshape(EP, 1)
    at3 = jnp.pad(atom_types, (0, NP - N)).reshape(NB, 1, 512)
    gid3 = jnp.pad(graph_ids, (0, NP - N), constant_values=ngraph
                   ).reshape(NB, 1, 512)
    embp = jnp.pad(emb, ((0, 128 - emb.shape[0]), (0, 0)))
    zrows = jnp.zeros((ZR, DIM), jnp.float32)
    fcaux = jnp.zeros((8, 128), jnp.float32)
    fcaux = fcaux.at[0, :].set(bfc).at[1, :].set(Wout[:, 0]).at[2, 0].set(bout[0])

    node = None
    new_node = None
    for i in range(NCONV):
        wab = jnp.concatenate([W1[i, :DIM, :], W1[i, DIM:2 * DIM, :]], axis=1)
        wc = W1[i, 2 * DIM:, :]
        pb = jnp.stack([b1[i], bn1_g[i], bn1_b[i], bn2_g[i], bn2_b[i],
                        jnp.zeros_like(b1[i]), jnp.zeros_like(b1[i]),
                        jnp.zeros_like(b1[i])])
        if i == 0:
            node, pa, pbj = _tc_first(at3, embp, wab)
        else:
            p3 = jnp.stack([bn3_g[i - 1], bn3_b[i - 1]] + [jnp.zeros_like(b1[0])] * 6)
            node, pa, pbj = _tc_update_proj(node, new_node, wab, p3, n_true)
        g = _sc_gather_add(pa, pbj, src, dst)
        z, st1 = _tc_zstats(g, dist3, wc, pb, e_true)
        st2 = _tc_fstats(z, st1, pb, e_true)
        h = _tc_h(z, st1, st2, pb, e_true)
        new_node = _sc_scatter_add(h, dst, zrows)

    p3 = jnp.stack([bn3_g[NCONV - 1], bn3_b[NCONV - 1]]
                   + [jnp.zeros_like(b1[0])] * 6)
    return _tc_readout(node, new_node, gid3, p3, Wfc, fcaux, n_true, ngraph)


# speedup vs baseline: 1.7178x; 1.0900x over previous
import functools

import jax
import jax.numpy as jnp
from jax import lax
from jax.experimental import pallas as pl
from jax.experimental.pallas import tpu as pltpu
from jax.experimental.pallas import tpu_sc as plsc

DIM = 64
NG = 64
CUTOFF = 12.0
NCONV = 3

NP = 50176
EP = 819200
NB = NP // 512
EB = 2048
NEB = EP // EB
HALF = NP // 2
SROWS = 25120
TRASH = HALF
ZR = 1570

_sc_mesh = lambda: plsc.VectorSubcoreMesh(core_axis_name="c", subcore_axis_name="s",
                                          num_cores=2, num_subcores=16)


def _sc_gather_add(pa, pb, src, dst):
    ne = src.shape[0]
    per_w = ne // 32
    CH = 256
    Q = CH // 128
    nch = per_w // CH

    @functools.partial(
        pl.kernel,
        out_type=jax.ShapeDtypeStruct((ne, DIM), jnp.bfloat16),
        mesh=_sc_mesh(),
        scratch_types=[
            [pltpu.VMEM((CH,), jnp.int32)] * 2,
            [pltpu.VMEM((CH,), jnp.int32)] * 2,
            [pltpu.VMEM((CH, DIM), jnp.bfloat16)] * 2,
            [pltpu.VMEM((CH, DIM), jnp.bfloat16)] * 2,
            [pltpu.SemaphoreType.DMA] * 2,
            [pltpu.SemaphoreType.DMA] * 2,
            [pltpu.SemaphoreType.DMA] * 2,
        ],
        compiler_params=pltpu.CompilerParams(use_tc_tiling_on_sc=False),
    )
    def k(pa_hbm, pb_hbm, src_hbm, dst_hbm, g_hbm, ia, ib, ra, rb, si, sg, ss):
        wid = lax.axis_index("s") * 2 + lax.axis_index("c")
        base = wid * per_w

        def fire_idx(kb, ci):
            off = base + ci * CH
            pltpu.async_copy(src_hbm.at[pl.ds(off, CH)], ia[kb], si[kb])
            pltpu.async_copy(dst_hbm.at[pl.ds(off, CH)], ib[kb], si[kb])

        def wait_idx(kb, ci):
            off = base + ci * CH
            pltpu.make_async_copy(src_hbm.at[pl.ds(off, CH)], ia[kb], si[kb]).wait()
            pltpu.make_async_copy(dst_hbm.at[pl.ds(off, CH)], ib[kb], si[kb]).wait()

        def fire_gather(kb):
            for q in range(Q):
                s = pl.ds(q * 128, 128)
                pltpu.async_copy(pa_hbm.at[ia[kb].at[s]], ra[kb].at[s], sg[kb])
                pltpu.async_copy(pb_hbm.at[ib[kb].at[s]], rb[kb].at[s], sg[kb])

        def wait_gather(kb):
            for q in range(Q):
                s = pl.ds(q * 128, 128)
                pltpu.make_async_copy(pa_hbm.at[ia[kb].at[s]], ra[kb].at[s],
                                      sg[kb]).wait()
                pltpu.make_async_copy(pb_hbm.at[ib[kb].at[s]], rb[kb].at[s],
                                      sg[kb]).wait()

        def fire_store(kb, ci):
            off = base + ci * CH
            pltpu.async_copy(ra[kb], g_hbm.at[pl.ds(off, CH)], ss[kb])

        def wait_store(kb, ci):
            off = base + ci * CH
            pltpu.make_async_copy(ra[kb], g_hbm.at[pl.ds(off, CH)], ss[kb]).wait()

        fire_idx(0, 0)
        wait_idx(0, 0)
        fire_gather(0)
        fire_idx(1, 1)

        @pl.loop(0, nch, step=2)
        def _(ci):
            for u in range(2):
                kb = u
                ko = 1 - u
                ck = ci + u
                wait_gather(kb)

                @pl.when(ck + 2 < nch)
                def _():
                    fire_idx(kb, ck + 2)

                @pl.when(ck >= 1)
                def _():
                    wait_store(ko, ck - 1)

                @pl.when(ck + 1 < nch)
                def _():
                    wait_idx(ko, ck + 1)
                    fire_gather(ko)

                @pl.loop(0, CH, step=4)
                def _(r):
                    for v in range(4):
                        for colq in range(2):
                            s = pl.ds(colq * 32, 32)
                            ra[kb][r + v, s] = ra[kb][r + v, s] + rb[kb][r + v, s]

                fire_store(kb, ck)

        wait_store(1, nch - 1)

    return k(pa, pb, src, dst)


def _sc_scatter_add(h, dst, init):
    ne = h.shape[0]
    per_t = ne // 16
    C = 128
    nch = per_t // C
    wo = HALF // 16
    TAIL = HALF - 15 * ZR

    @functools.partial(
        pl.kernel,
        out_type=jax.ShapeDtypeStruct((NP, DIM), jnp.float32),
        mesh=_sc_mesh(),
        scratch_types=[
            [pltpu.VMEM((C, DIM), jnp.float32)] * 2,
            [pltpu.VMEM((C,), jnp.int32)] * 2,
            [pltpu.VMEM((C // 128, 128), jnp.int32)] * 2,
            pltpu.VMEM_SHARED((SROWS, DIM), jnp.float32),
            [pltpu.SemaphoreType.DMA] * 2,
            [pltpu.SemaphoreType.DMA] * 2,
        ],
        compiler_params=pltpu.CompilerParams(use_tc_tiling_on_sc=False),
    )
    def k(h_hbm, dst_hbm, init_hbm, out_hbm, rows, di, li, acc, sl, sa):
        c = lax.axis_index("c")
        s = lax.axis_index("s")
        QS = C // 128
        @pl.when(s < 15)
        def _():
            pltpu.sync_copy(init_hbm.at[pl.ds(c * HALF + s * ZR, ZR)],
                            acc.at[pl.ds(s * ZR, ZR)])

        @pl.when(s == 15)
        def _():
            pltpu.sync_copy(init_hbm.at[pl.ds(c * HALF + 15 * ZR, TAIL)],
                            acc.at[pl.ds(15 * ZR, TAIL)])

        plsc.subcore_barrier()
        base_node = c * HALF

        def fire_load(kb, ci):
            off = s * per_t + ci * C
            pltpu.async_copy(dst_hbm.at[pl.ds(off, C)], di[kb], sl[kb])
            pltpu.async_copy(h_hbm.at[pl.ds(off, C)], rows[kb], sl[kb])

        def wait_load(kb, ci):
            off = s * per_t + ci * C
            pltpu.make_async_copy(dst_hbm.at[pl.ds(off, C)], di[kb], sl[kb]).wait()
            pltpu.make_async_copy(h_hbm.at[pl.ds(off, C)], rows[kb], sl[kb]).wait()

        def fire_scatter(kb):
            for q in range(QS):
                pltpu.async_copy(rows[kb].at[pl.ds(q * 128, 128)],
                                 acc.at[li[kb].at[q]], sa[kb], add=True)

        def wait_scatter(kb):
            for q in range(QS):
                pltpu.make_async_copy(rows[kb].at[pl.ds(q * 128, 128)],
                                      acc.at[li[kb].at[q]], sa[kb]).wait()

        fire_load(0, 0)

        @pl.loop(0, nch, step=2)
        def _(ci):
            for u in range(2):
                kb = u
                ko = 1 - u
                ck = ci + u
                wait_load(kb, ck)
                for q in range(QS):
                    @pl.loop(0, 128, step=16)
                    def _(l):
                        v = di[kb][pl.ds(q * 128 + l, 16)] - base_node
                        ok = (v >= 0) & (v < HALF)
                        li[kb][q, pl.ds(l, 16)] = jnp.where(ok, v, TRASH)

                @pl.when(ck >= 1)
                def _():
                    wait_scatter(ko)

                @pl.when(ck + 1 < nch)
                def _():
                    fire_load(ko, ck + 1)

                fire_scatter(kb)

        wait_scatter(1)
        plsc.subcore_barrier()
        pltpu.sync_copy(acc.at[pl.ds(s * wo, wo)],
                        out_hbm.at[pl.ds(c * HALF + s * wo, wo)])

    return k(h, dst, init)


def _tc_first(at3, embp, wab):
    def body(at_ref, emb_ref, w_ref, node_ref, pa_ref, pb_ref):
        at = at_ref[0, 0, :]
        oh = (at[:, None] == lax.broadcasted_iota(jnp.int32, (512, 128), 1)
              ).astype(jnp.float32)
        node = jnp.dot(oh, emb_ref[...], preferred_element_type=jnp.float32)
        node_ref[...] = node
        proj = jnp.dot(node, w_ref[...], preferred_element_type=jnp.float32)
        pa_ref[...] = proj[:, :DIM].astype(jnp.bfloat16)
        pb_ref[...] = proj[:, DIM:].astype(jnp.bfloat16)

    return pl.pallas_call(
        body,
        grid=(NB,),
        in_specs=[
            pl.BlockSpec((1, 1, 512), lambda j: (j, 0, 0)),
            pl.BlockSpec((128, DIM), lambda j: (0, 0)),
            pl.BlockSpec((DIM, 2 * DIM), lambda j: (0, 0)),
        ],
        out_specs=[
            pl.BlockSpec((512, DIM), lambda j: (j, 0)),
            pl.BlockSpec((512, DIM), lambda j: (j, 0)),
            pl.BlockSpec((512, DIM), lambda j: (j, 0)),
        ],
        out_shape=[
            jax.ShapeDtypeStruct((NP, DIM), jnp.float32),
            jax.ShapeDtypeStruct((NP, DIM), jnp.bfloat16),
            jax.ShapeDtypeStruct((NP, DIM), jnp.bfloat16),
        ],
    )(at3, embp, wab)


def _tc_update_proj(node_prev, new_node, wab, p3, n_true):
    def body(np_ref, nn_ref, w_ref, p3_ref, node_ref, pa_ref, pb_ref, acc):
        p = pl.program_id(0)
        j = pl.program_id(1)

        @pl.when((p == 0) & (j == 0))
        def _():
            acc[...] = jnp.zeros_like(acc)

        @pl.when(p == 0)
        def _():
            x = nn_ref[...]
            row = lax.broadcasted_iota(jnp.int32, (512, 1), 0) + j * 512
            xm = jnp.where(row < n_true, x, 0.0)
            acc[0:1, :] += jnp.sum(xm, axis=0, keepdims=True)
            acc[1:2, :] += jnp.sum(xm * xm, axis=0, keepdims=True)

        @pl.when(p == 1)
        def _():
            mu = acc[0:1, :] / n_true
            var = acc[1:2, :] / n_true - mu * mu
            rstd = lax.rsqrt(var + 1e-5)
            node = np_ref[...] + (nn_ref[...] - mu) * rstd * p3_ref[0:1, :] \
                + p3_ref[1:2, :]
            node_ref[...] = node
            proj = jnp.dot(node, w_ref[...], preferred_element_type=jnp.float32)
            pa_ref[...] = proj[:, :DIM].astype(jnp.bfloat16)
            pb_ref[...] = proj[:, DIM:].astype(jnp.bfloat16)

    return pl.pallas_call(
        body,
        grid=(2, NB),
        in_specs=[
            pl.BlockSpec((512, DIM), lambda p, j: (j, 0)),
            pl.BlockSpec((512, DIM), lambda p, j: (j, 0)),
            pl.BlockSpec((DIM, 2 * DIM), lambda p, j: (0, 0)),
            pl.BlockSpec((8, DIM), lambda p, j: (0, 0)),
        ],
        out_specs=[
            pl.BlockSpec((512, DIM), lambda p, j: (j, 0)),
            pl.BlockSpec((512, DIM), lambda p, j: (j, 0)),
            pl.BlockSpec((512, DIM), lambda p, j: (j, 0)),
        ],
        out_shape=[
            jax.ShapeDtypeStruct((NP, DIM), jnp.float32),
            jax.ShapeDtypeStruct((NP, DIM), jnp.bfloat16),
            jax.ShapeDtypeStruct((NP, DIM), jnp.bfloat16),
        ],
        scratch_shapes=[pltpu.VMEM((8, DIM), jnp.float32)],
    )(node_prev, new_node, wab, p3)


def _tc_zstats(g, dist3, wc, pb, e_true, row0):
    delta = CUTOFF / (NG - 1)
    nblk = g.shape[0] // EB

    def body(g_ref, d_ref, wc_ref, pb_ref, z_ref, st_ref):
        j = pl.program_id(0)
        d = d_ref[...]
        cent = lax.broadcasted_iota(jnp.int32, (1, NG), 1).astype(jnp.float32) * delta
        rbf = jnp.exp(-(((d - cent) / delta) ** 2))
        z = g_ref[...].astype(jnp.float32) + jnp.dot(rbf, wc_ref[...],
                                 preferred_element_type=jnp.float32) \
            + pb_ref[0:1, :]
        z_ref[...] = z
        row = lax.broadcasted_iota(jnp.int32, (EB, 1), 0) + j * EB + row0
        zm = jnp.where(row < e_true, z, 0.0)

        @pl.when(j == 0)
        def _():
            st_ref[...] = jnp.zeros_like(st_ref)

        st_ref[0:1, :] += jnp.sum(zm, axis=0, keepdims=True)
        st_ref[1:2, :] += jnp.sum(zm * zm, axis=0, keepdims=True)

    return pl.pallas_call(
        body,
        grid=(nblk,),
        in_specs=[
            pl.BlockSpec((EB, DIM), lambda j: (j, 0)),
            pl.BlockSpec((EB, 1), lambda j: (j, 0)),
            pl.BlockSpec((NG, DIM), lambda j: (0, 0)),
            pl.BlockSpec((8, DIM), lambda j: (0, 0)),
        ],
        out_specs=[
            pl.BlockSpec((EB, DIM), lambda j: (j, 0)),
            pl.BlockSpec((8, DIM), lambda j: (0, 0)),
        ],
        out_shape=[
            jax.ShapeDtypeStruct((g.shape[0], DIM), jnp.float32),
            jax.ShapeDtypeStruct((8, DIM), jnp.float32),
        ],
    )(g, dist3, wc, pb)


def _f_of_z(z, st1_ref, pb_ref, e_true):
    mu = st1_ref[0:1, :] / e_true
    var = st1_ref[1:2, :] / e_true - mu * mu
    rstd = lax.rsqrt(var + 1e-5)
    return jax.nn.sigmoid((z - mu) * rstd * pb_ref[1:2, :] + pb_ref[2:3, :])


def _tc_fstats(z, st1, pb, e_true, row0):
    def body(z_ref, st1_ref, pb_ref, st_ref):
        j = pl.program_id(0)
        f = _f_of_z(z_ref[...], st1_ref, pb_ref, e_true)
        row = lax.broadcasted_iota(jnp.int32, (EB, 1), 0) + j * EB + row0
        fm = jnp.where(row < e_true, f, 0.0)

        @pl.when(j == 0)
        def _():
            st_ref[...] = jnp.zeros_like(st_ref)

        st_ref[0:1, :] += jnp.sum(fm, axis=0, keepdims=True)
        st_ref[1:2, :] += jnp.sum(fm * fm, axis=0, keepdims=True)

    return pl.pallas_call(
        body,
        grid=(z.shape[0] // EB,),
        in_specs=[
            pl.BlockSpec((EB, DIM), lambda j: (j, 0)),
            pl.BlockSpec((8, DIM), lambda j: (0, 0)),
            pl.BlockSpec((8, DIM), lambda j: (0, 0)),
        ],
        out_specs=[pl.BlockSpec((8, DIM), lambda j: (0, 0))],
        out_shape=[jax.ShapeDtypeStruct((8, DIM), jnp.float32)],
    )(z, st1, pb)[0]


def _tc_h(z, st1, st2, pb, e_true, row0):
    def body(z_ref, st1_ref, st2_ref, pb_ref, h_ref):
        j = pl.program_id(0)
        f = _f_of_z(z_ref[...], st1_ref, pb_ref, e_true)
        mu = st2_ref[0:1, :] / e_true
        var = st2_ref[1:2, :] / e_true - mu * mu
        rstd = lax.rsqrt(var + 1e-5)
        c = jax.nn.softplus((f - mu) * rstd * pb_ref[3:4, :] + pb_ref[4:5, :])
        h = f * c
        row = lax.broadcasted_iota(jnp.int32, (EB, 1), 0) + j * EB + row0
        h_ref[...] = jnp.where(row < e_true, h, 0.0)

    return pl.pallas_call(
        body,
        grid=(z.shape[0] // EB,),
        in_specs=[
            pl.BlockSpec((EB, DIM), lambda j: (j, 0)),
            pl.BlockSpec((8, DIM), lambda j: (0, 0)),
            pl.BlockSpec((8, DIM), lambda j: (0, 0)),
            pl.BlockSpec((8, DIM), lambda j: (0, 0)),
        ],
        out_specs=[pl.BlockSpec((EB, DIM), lambda j: (j, 0))],
        out_shape=[jax.ShapeDtypeStruct((z.shape[0], DIM), jnp.float32)],
    )(z, st1, st2, pb)[0]


def _tc_readout(node_prev, new_node, gid3, p3, wfc, fcaux, n_true, ngraph):
    def body(np_ref, nn_ref, gid_ref, p3_ref, wfc_ref, aux_ref, out_ref,
             acc, gsum, gcnt):
        p = pl.program_id(0)
        j = pl.program_id(1)

        @pl.when((p == 0) & (j == 0))
        def _():
            acc[...] = jnp.zeros_like(acc)
            gsum[...] = jnp.zeros_like(gsum)
            gcnt[...] = jnp.zeros_like(gcnt)

        @pl.when(p == 0)
        def _():
            x = nn_ref[...]
            row = lax.broadcasted_iota(jnp.int32, (512, 1), 0) + j * 512
            xm = jnp.where(row < n_true, x, 0.0)
            acc[0:1, :] += jnp.sum(xm, axis=0, keepdims=True)
            acc[1:2, :] += jnp.sum(xm * xm, axis=0, keepdims=True)

        @pl.when(p == 1)
        def _():
            mu = acc[0:1, :] / n_true
            var = acc[1:2, :] / n_true - mu * mu
            rstd = lax.rsqrt(var + 1e-5)
            node = np_ref[...] + (nn_ref[...] - mu) * rstd * p3_ref[0:1, :] \
                + p3_ref[1:2, :]
            gid = gid_ref[0, 0, :]
            oh = (gid[:, None] == lax.broadcasted_iota(
                jnp.int32, (512, ngraph), 1)).astype(jnp.float32)
            gsum[...] += lax.dot_general(
                oh, node, (((0,), (0,)), ((), ())),
                preferred_element_type=jnp.float32)
            gcnt[0:1, :] += jnp.sum(oh, axis=0, keepdims=True)

        @pl.when((p == 1) & (j == NB - 1))
        def _():
            cnt = jnp.transpose(gcnt[0:1, :], (1, 0))
            crys = gsum[...] / jnp.maximum(cnt, 1.0)
            a1 = jnp.dot(jax.nn.softplus(crys), wfc_ref[...],
                         preferred_element_type=jnp.float32) + aux_ref[0:1, :]
            a2 = jax.nn.softplus(a1)
            res = jnp.sum(a2 * aux_ref[1:2, :], axis=1, keepdims=True) \
                + aux_ref[2, 0]
            out_ref[...] = res

    return pl.pallas_call(
        body,
        grid=(2, NB),
        in_specs=[
            pl.BlockSpec((512, DIM), lambda p, j: (j, 0)),
            pl.BlockSpec((512, DIM), lambda p, j: (j, 0)),
            pl.BlockSpec((1, 1, 512), lambda p, j: (j, 0, 0)),
            pl.BlockSpec((8, DIM), lambda p, j: (0, 0)),
            pl.BlockSpec((DIM, 128), lambda p, j: (0, 0)),
            pl.BlockSpec((8, 128), lambda p, j: (0, 0)),
        ],
        out_specs=[pl.BlockSpec((ngraph, 1), lambda p, j: (0, 0))],
        out_shape=[jax.ShapeDtypeStruct((ngraph, 1), jnp.float32)],
        scratch_shapes=[
            pltpu.VMEM((8, DIM), jnp.float32),
            pltpu.VMEM((ngraph, DIM), jnp.float32),
            pltpu.VMEM((8, ngraph), jnp.float32),
        ],
    )(node_prev, new_node, gid3, p3, wfc, fcaux)[0]


def kernel(distance, edge_index, atom_types, graph_ids, emb, W1, b1, W2, b2,
           bn1_g, bn1_b, bn2_g, bn2_b, bn3_g, bn3_b, Wfc, bfc, Wout, bout):
    E = distance.shape[0]
    N = atom_types.shape[0]
    ngraph = 512
    e_true = float(E)
    n_true = float(N)

    EPH = EP // 2
    src = jnp.pad(edge_index[0], (0, EP - E))
    dst = jnp.pad(edge_index[1], (0, EP - E))
    dist3 = jnp.pad(distance, (0, EP - E)).reshape(EP, 1)
    srcs = (src[:EPH], src[EPH:])
    dsts = (dst[:EPH], dst[EPH:])
    dists = (dist3[:EPH], dist3[EPH:])
    zinit = jnp.zeros((NP, DIM), jnp.float32)
    at3 = jnp.pad(atom_types, (0, NP - N)).reshape(NB, 1, 512)
    gid3 = jnp.pad(graph_ids, (0, NP - N), constant_values=ngraph
                   ).reshape(NB, 1, 512)
    embp = jnp.pad(emb, ((0, 128 - emb.shape[0]), (0, 0)))
    fcaux = jnp.zeros((8, 128), jnp.float32)
    fcaux = fcaux.at[0, :].set(bfc).at[1, :].set(Wout[:, 0]).at[2, 0].set(bout[0])

    node = None
    new_node = None
    for i in range(NCONV):
        wab = jnp.concatenate([W1[i, :DIM, :], W1[i, DIM:2 * DIM, :]], axis=1)
        wc = W1[i, 2 * DIM:, :]
        pb = jnp.stack([b1[i], bn1_g[i], bn1_b[i], bn2_g[i], bn2_b[i],
                        jnp.zeros_like(b1[i]), jnp.zeros_like(b1[i]),
                        jnp.zeros_like(b1[i])])
        if i == 0:
            node, pa, pbj = _tc_first(at3, embp, wab)
        else:
            p3 = jnp.stack([bn3_g[i - 1], bn3_b[i - 1]] + [jnp.zeros_like(b1[0])] * 6)
            node, pa, pbj = _tc_update_proj(node, new_node, wab, p3, n_true)
        g1 = _sc_gather_add(pa, pbj, srcs[0], dsts[0])
        g2 = _sc_gather_add(pa, pbj, srcs[1], dsts[1])
        z1, s1a = _tc_zstats(g1, dists[0], wc, pb, e_true, 0)
        z2, s1b = _tc_zstats(g2, dists[1], wc, pb, e_true, EPH)
        st1 = s1a + s1b
        st2 = _tc_fstats(z1, st1, pb, e_true, 0) \
            + _tc_fstats(z2, st1, pb, e_true, EPH)
        h1 = _tc_h(z1, st1, st2, pb, e_true, 0)
        h2 = _tc_h(z2, st1, st2, pb, e_true, EPH)
        nn1 = _sc_scatter_add(h1, dsts[0], zinit)
        new_node = _sc_scatter_add(h2, dsts[1], nn1)

    p3 = jnp.stack([bn3_g[NCONV - 1], bn3_b[NCONV - 1]]
                   + [jnp.zeros_like(b1[0])] * 6)
    return _tc_readout(node, new_node, gid3, p3, Wfc, fcaux, n_true, ngraph)
